# baseline (speedup 1.0000x reference)
import functools

import jax
import jax.numpy as jnp
from jax import lax
from jax.experimental import pallas as pl
from jax.experimental.pallas import tpu as pltpu
from jax.experimental.pallas import tpu_sc as plsc

_WIN = 128
_OVERLAP = 0


def kernel(x):
    B, T = x.shape
    stride = _WIN - _OVERLAP
    n_windows = T // _WIN

    nc = 1
    rows_per_core = B // nc

    mesh = plsc.ScalarSubcoreMesh(axis_name="c", num_cores=nc)

    @functools.partial(
        pl.kernel,
        mesh=mesh,
        out_type=jax.ShapeDtypeStruct((B * n_windows * _WIN,), x.dtype),
        scratch_types=[
            pltpu.VMEM_SHARED((B, T), x.dtype),
            [pltpu.SemaphoreType.DMA] * B,
            pltpu.SemaphoreType.DMA,
        ],
    )
    def _rolling_window(x_hbm, out_hbm, buf, sems_in, sem_out):
        ins = []
        outs = []
        for b in range(B):
            src = x_hbm.at[b, pl.ds(0, n_windows * stride)]
            ins.append(pltpu.make_async_copy(src, buf.at[b], sems_in[b]))
            dst = out_hbm.at[pl.ds(b * n_windows * _WIN, n_windows * _WIN)]
            outs.append(pltpu.make_async_copy(buf.at[b], dst, sem_out))
        for c in ins:
            c.start()
        for b in range(B):
            ins[b].wait()
            outs[b].start()
        pltpu.make_async_copy(out_hbm, out_hbm, sem_out).wait()

    out_flat = _rolling_window(x)
    return out_flat.reshape(B, n_windows, _WIN)


# speedup vs baseline: 1.0016x; 1.0016x over previous
import functools

import jax
import jax.numpy as jnp
from jax import lax
from jax.experimental import pallas as pl
from jax.experimental.pallas import tpu as pltpu
from jax.experimental.pallas import tpu_sc as plsc

_WIN = 128
_OVERLAP = 0


def kernel(x):
    B, T = x.shape
    stride = _WIN - _OVERLAP
    n_windows = T // _WIN

    nc = 1
    rows_per_core = B // nc

    mesh = plsc.ScalarSubcoreMesh(axis_name="c", num_cores=nc)

    @functools.partial(
        pl.kernel,
        mesh=mesh,
        out_type=jax.ShapeDtypeStruct((B * n_windows * _WIN,), x.dtype),
        scratch_types=[
            pltpu.VMEM_SHARED((B * T,), x.dtype),
            pltpu.SemaphoreType.DMA,
            pltpu.SemaphoreType.DMA,
        ],
    )
    def _rolling_window(x_hbm, out_hbm, buf, sem_in, sem_out):
        ins = []
        for b in range(B):
            src = x_hbm.at[b, pl.ds(0, n_windows * stride)]
            dst = buf.at[pl.ds(b * n_windows * _WIN, n_windows * _WIN)]
            ins.append(pltpu.make_async_copy(src, dst, sem_in))
        for c in ins:
            c.start()
        pltpu.make_async_copy(buf, buf, sem_in).wait()
        out_c = pltpu.make_async_copy(buf, out_hbm, sem_out)
        out_c.start()
        out_c.wait()

    out_flat = _rolling_window(x)
    return out_flat.reshape(B, n_windows, _WIN)
